# Initial kernel scaffold; baseline (speedup 1.0000x reference)
import jax
import jax.numpy as jnp
from jax.experimental import pallas as pl


def kernel(tokens, table):
    raise NotImplementedError("write your pallas kernel here")


# speedup vs baseline: 1.4175x; 1.4175x over previous
import functools
import math

import jax
import jax.numpy as jnp
from jax import lax
from jax.experimental import pallas as pl
from jax.experimental.pallas import tpu as pltpu
from jax.experimental.pallas import tpu_sc as plsc

EMB_D = 32
LANES = 16
NUM_CORES = 2
NUM_SUBCORES = 16
NUM_WORKERS = NUM_CORES * NUM_SUBCORES

CHUNK = 1600


def _body(table_hbm, idx_hbm, out_hbm, idx_v, rows_v, sem):
    wid = lax.axis_index("s") * NUM_CORES + lax.axis_index("c")
    b_total = idx_hbm.shape[0]
    b_per_w = b_total // NUM_WORKERS
    n_chunks = b_per_w // CHUNK
    base = wid * b_per_w
    scale = jnp.float32(math.sqrt(EMB_D))

    def chunk_step(g, _):
        off = base + g * CHUNK
        pltpu.sync_copy(idx_hbm.at[pl.ds(off, CHUNK)], idx_v)
        pltpu.async_copy(table_hbm.at[idx_v], rows_v, sem).wait()

        def scale_step(r, _):
            row = r * 4
            for u in range(4):
                for j in range(EMB_D // LANES):
                    sl = pl.ds(j * LANES, LANES)
                    rows_v[row + u, sl] = rows_v[row + u, sl] * scale
            return 0

        lax.fori_loop(0, CHUNK // 4, scale_step, 0)
        pltpu.sync_copy(rows_v, out_hbm.at[pl.ds(off, CHUNK)])
        return 0

    lax.fori_loop(0, n_chunks, chunk_step, 0)


def _gather_scaled(table, idx):
    b_total = idx.shape[0]
    mesh = plsc.VectorSubcoreMesh(core_axis_name="c", subcore_axis_name="s")
    k = functools.partial(
        pl.kernel,
        mesh=mesh,
        out_type=jax.ShapeDtypeStruct((b_total, EMB_D), jnp.float32),
        compiler_params=pltpu.CompilerParams(use_tc_tiling_on_sc=False),
        scratch_types=[
            pltpu.VMEM((CHUNK,), jnp.int32),
            pltpu.VMEM((CHUNK, EMB_D), jnp.float32),
            pltpu.SemaphoreType.DMA,
        ],
    )(_body)
    return k(table, idx)


def kernel(tokens, table):
    b_total = tokens.size
    idx = tokens.reshape(b_total).astype(jnp.int32)
    out = _gather_scaled(table, idx)
    return out.reshape(*tokens.shape, EMB_D)


# speedup vs baseline: 1.4789x; 1.0433x over previous
import functools
import math

import jax
import jax.numpy as jnp
from jax import lax
from jax.experimental import pallas as pl
from jax.experimental.pallas import tpu as pltpu
from jax.experimental.pallas import tpu_sc as plsc

EMB_D = 32
LANES = 16
NUM_CORES = 2
NUM_SUBCORES = 16
NUM_WORKERS = NUM_CORES * NUM_SUBCORES

CHUNK = 800
NBUF = 2
ROWS_UNROLL = 4


def _scale_chunk(src_v, dst_v, scale):
    def scale_step(r, _):
        row = r * ROWS_UNROLL
        for u in range(ROWS_UNROLL):
            for j in range(EMB_D // LANES):
                sl = pl.ds(j * LANES, LANES)
                dst_v[row + u, sl] = src_v[row + u, sl] * scale
        return 0

    lax.fori_loop(0, CHUNK // ROWS_UNROLL, scale_step, 0)


def _body(table_hbm, idx_hbm, out_hbm,
          idx0, idx1, in0, in1, out0, out1,
          gsem0, gsem1, ssem0, ssem1, isem0, isem1):
    wid = lax.axis_index("s") * NUM_CORES + lax.axis_index("c")
    b_total = idx_hbm.shape[0]
    b_per_w = b_total // NUM_WORKERS
    n_chunks = b_per_w // CHUNK
    base = wid * b_per_w
    scale = jnp.float32(math.sqrt(EMB_D))

    slots = (
        (idx0, in0, out0, gsem0, ssem0, isem0),
        (idx1, in1, out1, gsem1, ssem1, isem1),
    )

    for b in range(NBUF):
        idx_v, in_v, _, gsem, _, _ = slots[b]
        off = base + b * CHUNK
        pltpu.sync_copy(idx_hbm.at[pl.ds(off, CHUNK)], idx_v)
        pltpu.async_copy(table_hbm.at[idx_v], in_v, gsem)

    def outer(i, _):
        for b in range(NBUF):
            idx_v, in_v, out_v, gsem, ssem, isem = slots[b]
            g = i * NBUF + b
            off = base + g * CHUNK
            nxt = g + NBUF

            pltpu.make_async_copy(table_hbm.at[idx_v], in_v, gsem).wait()

            @pl.when(nxt < n_chunks)
            def _():
                noff = base + nxt * CHUNK
                pltpu.async_copy(idx_hbm.at[pl.ds(noff, CHUNK)], idx_v, isem)

            @pl.when(g >= NBUF)
            def _():
                poff = base + (g - NBUF) * CHUNK
                pltpu.make_async_copy(
                    out_v, out_hbm.at[pl.ds(poff, CHUNK)], ssem).wait()

            _scale_chunk(in_v, out_v, scale)

            pltpu.async_copy(out_v, out_hbm.at[pl.ds(off, CHUNK)], ssem)

            @pl.when(nxt < n_chunks)
            def _():
                noff = base + nxt * CHUNK
                pltpu.make_async_copy(
                    idx_hbm.at[pl.ds(noff, CHUNK)], idx_v, isem).wait()
                pltpu.async_copy(table_hbm.at[idx_v], in_v, gsem)

        return 0

    lax.fori_loop(0, n_chunks // NBUF, outer, 0)

    for b in range(NBUF):
        _, _, out_v, _, ssem, _ = slots[b]
        off = base + (n_chunks - NBUF + b) * CHUNK
        pltpu.make_async_copy(out_v, out_hbm.at[pl.ds(off, CHUNK)], ssem).wait()


def _gather_scaled(table, idx):
    b_total = idx.shape[0]
    mesh = plsc.VectorSubcoreMesh(core_axis_name="c", subcore_axis_name="s")
    k = functools.partial(
        pl.kernel,
        mesh=mesh,
        out_type=jax.ShapeDtypeStruct((b_total, EMB_D), jnp.float32),
        compiler_params=pltpu.CompilerParams(use_tc_tiling_on_sc=False),
        scratch_types=[
            pltpu.VMEM((CHUNK,), jnp.int32),
            pltpu.VMEM((CHUNK,), jnp.int32),
            pltpu.VMEM((CHUNK, EMB_D), jnp.float32),
            pltpu.VMEM((CHUNK, EMB_D), jnp.float32),
            pltpu.VMEM((CHUNK, EMB_D), jnp.float32),
            pltpu.VMEM((CHUNK, EMB_D), jnp.float32),
            pltpu.SemaphoreType.DMA,
            pltpu.SemaphoreType.DMA,
            pltpu.SemaphoreType.DMA,
            pltpu.SemaphoreType.DMA,
            pltpu.SemaphoreType.DMA,
            pltpu.SemaphoreType.DMA,
        ],
    )(_body)
    return k(table, idx)


def kernel(tokens, table):
    b_total = tokens.size
    idx = tokens.reshape(b_total).astype(jnp.int32)
    out = _gather_scaled(table, idx)
    return out.reshape(*tokens.shape, EMB_D)
